# baseline (speedup 1.0000x reference)
import functools

import jax
import jax.numpy as jnp
from jax.experimental import pallas as pl
from jax.experimental.pallas import tpu as pltpu

_NEG = float("-inf")


def _seg_prefix_max(y, gid2):
    b, c = y.shape
    pref = y
    s = 1
    while s < b:
        ysh = jnp.concatenate(
            [jnp.full((s, c), _NEG, jnp.float32), pref[:-s]], axis=0)
        gsh = jnp.concatenate(
            [jnp.full((s, 1), -1, jnp.int32), gid2[:-s]], axis=0)
        pen = jnp.where(gsh == gid2, 0.0, _NEG).astype(jnp.float32)
        pref = jnp.maximum(pref, ysh + pen)
        s *= 2
    return pref


def _seg_suffix_max(t, gid2):
    b, c = t.shape
    s = 1
    while s < b:
        ysh = jnp.concatenate(
            [t[s:], jnp.full((s, c), _NEG, jnp.float32)], axis=0)
        gsh = jnp.concatenate(
            [gid2[s:], jnp.full((s, 1), -1, jnp.int32)], axis=0)
        pen = jnp.where(gsh == gid2, 0.0, _NEG).astype(jnp.float32)
        t = jnp.maximum(t, ysh + pen)
        s *= 2
    return t


def _fwd_kernel(pf_ref, inv_ref, wt_ref, g_ref, b_ref, out_ref,
                vec_scr, gid_scr, *, hidden):
    i = pl.program_id(0)

    @pl.when(i == 0)
    def _():
        gid_scr[0] = -1
        vec_scr[...] = jnp.full_like(vec_scr[...], _NEG)

    pf = pf_ref[...]
    h = jnp.dot(pf, wt_ref[...], preferred_element_type=jnp.float32)
    mean = jnp.mean(h, axis=1, keepdims=True)
    cen = h - mean
    var = jnp.mean(cen * cen, axis=1, keepdims=True)
    yn = cen * jax.lax.rsqrt(var + 1e-5) * g_ref[...] + b_ref[...]
    y = yn * jax.nn.sigmoid(yn)

    gid2 = inv_ref[...]
    pref = _seg_prefix_max(y, gid2)
    carry_g = gid_scr[0]
    carry_v = vec_scr[...]
    g1 = jnp.where(gid2 == carry_g, jnp.maximum(pref, carry_v), pref)

    out_ref[:, :hidden] = y
    out_ref[:, hidden:] = g1

    gid_scr[0] = gid2[-1, 0]
    vec_scr[...] = g1[-1:, :]


def _bwd_kernel(z_ref, inv_ref, out_ref, vec_scr, gid_scr):
    i = pl.program_id(0)

    @pl.when(i == 0)
    def _():
        gid_scr[0] = -1
        vec_scr[...] = jnp.full_like(vec_scr[...], _NEG)

    g1 = z_ref[...]
    gid2 = inv_ref[...]
    suf = _seg_suffix_max(g1, gid2)
    res = jnp.where(gid2 == gid_scr[0], jnp.maximum(suf, vec_scr[...]), suf)
    out_ref[...] = res
    gid_scr[0] = gid2[0, 0]
    vec_scr[...] = res[:1, :]


def _pick_block(n):
    for b in (3200, 1600, 640, 512, 800, 256, 1000, 128, 200, 8):
        if n % b == 0:
            return b
    return n


def kernel(point_features, inverse, num_groups, W, gamma, beta):
    n, in_ch = point_features.shape
    hidden = W.shape[0]
    blk = _pick_block(n)
    nblk = n // blk

    inv2 = inverse.astype(jnp.int32).reshape(n, 1)
    wt = W.T
    g2 = gamma.reshape(1, hidden)
    b2 = beta.reshape(1, hidden)

    z = pl.pallas_call(
        functools.partial(_fwd_kernel, hidden=hidden),
        grid=(nblk,),
        in_specs=[
            pl.BlockSpec((blk, in_ch), lambda i: (i, 0)),
            pl.BlockSpec((blk, 1), lambda i: (i, 0)),
            pl.BlockSpec((in_ch, hidden), lambda i: (0, 0)),
            pl.BlockSpec((1, hidden), lambda i: (0, 0)),
            pl.BlockSpec((1, hidden), lambda i: (0, 0)),
        ],
        out_specs=pl.BlockSpec((blk, 2 * hidden), lambda i: (i, 0)),
        out_shape=jax.ShapeDtypeStruct((n, 2 * hidden), jnp.float32),
        scratch_shapes=[
            pltpu.VMEM((1, hidden), jnp.float32),
            pltpu.SMEM((1,), jnp.int32),
        ],
        compiler_params=pltpu.CompilerParams(
            dimension_semantics=("arbitrary",)),
    )(point_features, inv2, wt, g2, b2)

    out = pl.pallas_call(
        _bwd_kernel,
        grid=(nblk,),
        in_specs=[
            pl.BlockSpec((blk, hidden), lambda i, nb=nblk: (nb - 1 - i, 1)),
            pl.BlockSpec((blk, 1), lambda i, nb=nblk: (nb - 1 - i, 0)),
        ],
        out_specs=pl.BlockSpec((blk, hidden), lambda i, nb=nblk: (nb - 1 - i, 1)),
        out_shape=jax.ShapeDtypeStruct((n, 2 * hidden), jnp.float32),
        input_output_aliases={0: 0},
        scratch_shapes=[
            pltpu.VMEM((1, hidden), jnp.float32),
            pltpu.SMEM((1,), jnp.int32),
        ],
        compiler_params=pltpu.CompilerParams(
            dimension_semantics=("arbitrary",)),
    )(z, inv2)
    return out


# speedup vs baseline: 1.0530x; 1.0530x over previous
import functools

import jax
import jax.numpy as jnp
from jax.experimental import pallas as pl
from jax.experimental.pallas import tpu as pltpu

_NEG = float("-inf")


def _seg_prefix_max(y, gid2):
    b, c = y.shape
    pref = y
    s = 1
    while s < b:
        ysh = jnp.concatenate(
            [jnp.full((s, c), _NEG, jnp.float32), pref[:-s]], axis=0)
        gsh = jnp.concatenate(
            [jnp.full((s, 1), -1, jnp.int32), gid2[:-s]], axis=0)
        pen = jnp.where(gsh == gid2, 0.0, _NEG).astype(jnp.float32)
        pref = jnp.maximum(pref, ysh + pen)
        s *= 2
    return pref


def _seg_suffix_max(t, gid2):
    b, c = t.shape
    s = 1
    while s < b:
        ysh = jnp.concatenate(
            [t[s:], jnp.full((s, c), _NEG, jnp.float32)], axis=0)
        gsh = jnp.concatenate(
            [gid2[s:], jnp.full((s, 1), -1, jnp.int32)], axis=0)
        pen = jnp.where(gsh == gid2, 0.0, _NEG).astype(jnp.float32)
        t = jnp.maximum(t, ysh + pen)
        s *= 2
    return t


def _fwd_kernel(pf_ref, inv_ref, wt_ref, g_ref, b_ref, out_ref,
                vec_scr, gid_scr, *, hidden):
    i = pl.program_id(0)

    @pl.when(i == 0)
    def _():
        gid_scr[0] = -1
        vec_scr[...] = jnp.full_like(vec_scr[...], _NEG)

    pf = pf_ref[...]
    h = jnp.dot(pf, wt_ref[...], preferred_element_type=jnp.float32)
    mean = jnp.mean(h, axis=1, keepdims=True)
    cen = h - mean
    var = jnp.mean(cen * cen, axis=1, keepdims=True)
    yn = cen * jax.lax.rsqrt(var + 1e-5) * g_ref[...] + b_ref[...]
    y = yn * jax.nn.sigmoid(yn)

    gid2 = inv_ref[...]
    pref = _seg_prefix_max(y, gid2)
    carry_g = gid_scr[0]
    carry_v = vec_scr[...]
    g1 = jnp.where(gid2 == carry_g, jnp.maximum(pref, carry_v), pref)

    out_ref[:, :hidden] = y
    out_ref[:, hidden:] = g1

    gid_scr[0] = gid2[-1, 0]
    vec_scr[...] = g1[-1:, :]


def _bwd_kernel(z_ref, inv_ref, out_ref, vec_scr, gid_scr):
    i = pl.program_id(0)

    @pl.when(i == 0)
    def _():
        gid_scr[0] = -1
        vec_scr[...] = jnp.full_like(vec_scr[...], _NEG)

    g1 = z_ref[...]
    gid2 = inv_ref[...]
    suf = _seg_suffix_max(g1, gid2)
    res = jnp.where(gid2 == gid_scr[0], jnp.maximum(suf, vec_scr[...]), suf)
    out_ref[...] = res
    gid_scr[0] = gid2[0, 0]
    vec_scr[...] = res[:1, :]


def _pick_block(n):
    for b in (2000, 1600, 640, 512, 800, 256, 1000, 128, 200, 8):
        if n % b == 0:
            return b
    return n


def kernel(point_features, inverse, num_groups, W, gamma, beta):
    n, in_ch = point_features.shape
    hidden = W.shape[0]
    blk = _pick_block(n)
    nblk = n // blk

    inv2 = inverse.astype(jnp.int32).reshape(n, 1)
    wt = W.T
    g2 = gamma.reshape(1, hidden)
    b2 = beta.reshape(1, hidden)

    z = pl.pallas_call(
        functools.partial(_fwd_kernel, hidden=hidden),
        grid=(nblk,),
        in_specs=[
            pl.BlockSpec((blk, in_ch), lambda i: (i, 0)),
            pl.BlockSpec((blk, 1), lambda i: (i, 0)),
            pl.BlockSpec((in_ch, hidden), lambda i: (0, 0)),
            pl.BlockSpec((1, hidden), lambda i: (0, 0)),
            pl.BlockSpec((1, hidden), lambda i: (0, 0)),
        ],
        out_specs=pl.BlockSpec((blk, 2 * hidden), lambda i: (i, 0)),
        out_shape=jax.ShapeDtypeStruct((n, 2 * hidden), jnp.float32),
        scratch_shapes=[
            pltpu.VMEM((1, hidden), jnp.float32),
            pltpu.SMEM((1,), jnp.int32),
        ],
        compiler_params=pltpu.CompilerParams(
            dimension_semantics=("arbitrary",)),
    )(point_features, inv2, wt, g2, b2)

    out = pl.pallas_call(
        _bwd_kernel,
        grid=(nblk,),
        in_specs=[
            pl.BlockSpec((blk, hidden), lambda i, nb=nblk: (nb - 1 - i, 1)),
            pl.BlockSpec((blk, 1), lambda i, nb=nblk: (nb - 1 - i, 0)),
        ],
        out_specs=pl.BlockSpec((blk, hidden), lambda i, nb=nblk: (nb - 1 - i, 1)),
        out_shape=jax.ShapeDtypeStruct((n, 2 * hidden), jnp.float32),
        input_output_aliases={0: 0},
        scratch_shapes=[
            pltpu.VMEM((1, hidden), jnp.float32),
            pltpu.SMEM((1,), jnp.int32),
        ],
        compiler_params=pltpu.CompilerParams(
            dimension_semantics=("arbitrary",)),
    )(z, inv2)
    return out


# speedup vs baseline: 1.0546x; 1.0016x over previous
import functools

import jax
import jax.numpy as jnp
from jax.experimental import pallas as pl
from jax.experimental.pallas import tpu as pltpu

_NEG = float("-inf")


def _seg_prefix_max(y, gid2):
    b, c = y.shape
    pref = y
    s = 1
    while s < b:
        ysh = jnp.concatenate(
            [jnp.full((s, c), _NEG, jnp.float32), pref[:-s]], axis=0)
        gsh = jnp.concatenate(
            [jnp.full((s, 1), -1, jnp.int32), gid2[:-s]], axis=0)
        pen = jnp.where(gsh == gid2, 0.0, _NEG).astype(jnp.float32)
        pref = jnp.maximum(pref, ysh + pen)
        s *= 2
    return pref


def _seg_suffix_max(t, gid2):
    b, c = t.shape
    s = 1
    while s < b:
        ysh = jnp.concatenate(
            [t[s:], jnp.full((s, c), _NEG, jnp.float32)], axis=0)
        gsh = jnp.concatenate(
            [gid2[s:], jnp.full((s, 1), -1, jnp.int32)], axis=0)
        pen = jnp.where(gsh == gid2, 0.0, _NEG).astype(jnp.float32)
        t = jnp.maximum(t, ysh + pen)
        s *= 2
    return t


def _fwd_kernel(pf_ref, inv_ref, wt_ref, g_ref, b_ref, out_ref,
                vec_scr, gid_scr, *, hidden):
    i = pl.program_id(0)

    @pl.when(i == 0)
    def _():
        gid_scr[0] = -1
        vec_scr[...] = jnp.full_like(vec_scr[...], _NEG)

    pf = pf_ref[...]
    h = jnp.dot(pf, wt_ref[...], preferred_element_type=jnp.float32)
    mean = jnp.mean(h, axis=1, keepdims=True)
    cen = h - mean
    var = jnp.mean(cen * cen, axis=1, keepdims=True)
    yn = cen * jax.lax.rsqrt(var + 1e-5) * g_ref[...] + b_ref[...]
    y = yn * jax.nn.sigmoid(yn)

    gid2 = inv_ref[...]
    pref = _seg_prefix_max(y, gid2)
    carry_g = gid_scr[0]
    carry_v = vec_scr[...]
    g1 = jnp.where(gid2 == carry_g, jnp.maximum(pref, carry_v), pref)

    out_ref[:, :hidden] = y
    out_ref[:, hidden:] = g1

    gid_scr[0] = gid2[-1, 0]
    vec_scr[...] = g1[-1:, :]


def _bwd_kernel(z_ref, inv_ref, out_ref, vec_scr, gid_scr):
    i = pl.program_id(0)

    @pl.when(i == 0)
    def _():
        gid_scr[0] = -1
        vec_scr[...] = jnp.full_like(vec_scr[...], _NEG)

    g1 = z_ref[...]
    gid2 = inv_ref[...]
    suf = _seg_suffix_max(g1, gid2)
    res = jnp.where(gid2 == gid_scr[0], jnp.maximum(suf, vec_scr[...]), suf)
    out_ref[...] = res
    gid_scr[0] = gid2[0, 0]
    vec_scr[...] = res[:1, :]


def _pick_block(n):
    for b in (1600, 640, 512, 800, 256, 1000, 128, 200, 8):
        if n % b == 0:
            return b
    return n


def kernel(point_features, inverse, num_groups, W, gamma, beta):
    n, in_ch = point_features.shape
    hidden = W.shape[0]
    blk = _pick_block(n)
    nblk = n // blk

    inv2 = inverse.astype(jnp.int32).reshape(n, 1)
    wt = W.T
    g2 = gamma.reshape(1, hidden)
    b2 = beta.reshape(1, hidden)

    z = pl.pallas_call(
        functools.partial(_fwd_kernel, hidden=hidden),
        grid=(nblk,),
        in_specs=[
            pl.BlockSpec((blk, in_ch), lambda i: (i, 0)),
            pl.BlockSpec((blk, 1), lambda i: (i, 0)),
            pl.BlockSpec((in_ch, hidden), lambda i: (0, 0)),
            pl.BlockSpec((1, hidden), lambda i: (0, 0)),
            pl.BlockSpec((1, hidden), lambda i: (0, 0)),
        ],
        out_specs=pl.BlockSpec((blk, 2 * hidden), lambda i: (i, 0)),
        out_shape=jax.ShapeDtypeStruct((n, 2 * hidden), jnp.float32),
        scratch_shapes=[
            pltpu.VMEM((1, hidden), jnp.float32),
            pltpu.SMEM((1,), jnp.int32),
        ],
        compiler_params=pltpu.CompilerParams(
            dimension_semantics=("arbitrary",)),
    )(point_features, inv2, wt, g2, b2)

    out = pl.pallas_call(
        _bwd_kernel,
        grid=(nblk,),
        in_specs=[
            pl.BlockSpec((blk, hidden), lambda i, nb=nblk: (nb - 1 - i, 1)),
            pl.BlockSpec((blk, 1), lambda i, nb=nblk: (nb - 1 - i, 0)),
        ],
        out_specs=pl.BlockSpec((blk, hidden), lambda i, nb=nblk: (nb - 1 - i, 1)),
        out_shape=jax.ShapeDtypeStruct((n, 2 * hidden), jnp.float32),
        input_output_aliases={0: 0},
        scratch_shapes=[
            pltpu.VMEM((1, hidden), jnp.float32),
            pltpu.SMEM((1,), jnp.int32),
        ],
        compiler_params=pltpu.CompilerParams(
            dimension_semantics=("arbitrary",)),
    )(z, inv2)
    return out


# speedup vs baseline: 1.1068x; 1.0494x over previous
import functools

import jax
import jax.numpy as jnp
from jax.experimental import pallas as pl
from jax.experimental.pallas import tpu as pltpu

_NEG = float("-inf")


def _seg_prefix_max(y, gid2):
    b, c = y.shape
    pref = y
    s = 1
    while s < b:
        ysh = jnp.concatenate(
            [jnp.full((s, c), _NEG, jnp.float32), pref[:-s]], axis=0)
        gsh = jnp.concatenate(
            [jnp.full((s, 1), -1, jnp.int32), gid2[:-s]], axis=0)
        pen = jnp.where(gsh == gid2, 0.0, _NEG).astype(jnp.float32)
        pref = jnp.maximum(pref, ysh + pen)
        s *= 2
    return pref


def _seg_suffix_max(t, gid2):
    b, c = t.shape
    s = 1
    while s < b:
        ysh = jnp.concatenate(
            [t[s:], jnp.full((s, c), _NEG, jnp.float32)], axis=0)
        gsh = jnp.concatenate(
            [gid2[s:], jnp.full((s, 1), -1, jnp.int32)], axis=0)
        pen = jnp.where(gsh == gid2, 0.0, _NEG).astype(jnp.float32)
        t = jnp.maximum(t, ysh + pen)
        s *= 2
    return t


def _fwd_kernel(pf_ref, inv_ref, wt_ref, g_ref, b_ref, out_ref,
                vec_scr, gid_scr, *, hidden):
    i = pl.program_id(0)

    @pl.when(i == 0)
    def _():
        gid_scr[0] = -1
        vec_scr[...] = jnp.full_like(vec_scr[...], _NEG)

    pf = pf_ref[...]
    h = jnp.dot(pf, wt_ref[...], preferred_element_type=jnp.float32)
    mean = jnp.mean(h, axis=1, keepdims=True)
    cen = h - mean
    var = jnp.mean(cen * cen, axis=1, keepdims=True)
    yn = cen * jax.lax.rsqrt(var + 1e-5) * g_ref[...] + b_ref[...]
    y = yn * jax.nn.sigmoid(yn)

    gid2 = inv_ref[...]
    pref = _seg_prefix_max(y, gid2)
    carry_g = gid_scr[0]
    carry_v = vec_scr[...]
    g1 = jnp.where(gid2 == carry_g, jnp.maximum(pref, carry_v), pref)

    out_ref[:, :hidden] = y
    out_ref[:, hidden:] = g1

    gid_scr[0] = gid2[-1, 0]
    vec_scr[...] = g1[-1:, :]


def _bwd_kernel(z_ref, inv_ref, out_ref, vec_scr, gid_scr):
    i = pl.program_id(0)

    @pl.when(i == 0)
    def _():
        gid_scr[0] = -1
        vec_scr[...] = jnp.full_like(vec_scr[...], _NEG)

    g1 = z_ref[...]
    gid2 = inv_ref[...]
    suf = _seg_suffix_max(g1, gid2)
    res = jnp.where(gid2 == gid_scr[0], jnp.maximum(suf, vec_scr[...]), suf)
    out_ref[...] = res
    gid_scr[0] = gid2[0, 0]
    vec_scr[...] = res[:1, :]


def _pick_block(n):
    for b in (1000, 1600, 640, 512, 800, 256, 128, 200, 8):
        if n % b == 0:
            return b
    return n


def kernel(point_features, inverse, num_groups, W, gamma, beta):
    n, in_ch = point_features.shape
    hidden = W.shape[0]
    blk = _pick_block(n)
    nblk = n // blk

    inv2 = inverse.astype(jnp.int32).reshape(n, 1)
    wt = W.T
    g2 = gamma.reshape(1, hidden)
    b2 = beta.reshape(1, hidden)

    z = pl.pallas_call(
        functools.partial(_fwd_kernel, hidden=hidden),
        grid=(nblk,),
        in_specs=[
            pl.BlockSpec((blk, in_ch), lambda i: (i, 0)),
            pl.BlockSpec((blk, 1), lambda i: (i, 0)),
            pl.BlockSpec((in_ch, hidden), lambda i: (0, 0)),
            pl.BlockSpec((1, hidden), lambda i: (0, 0)),
            pl.BlockSpec((1, hidden), lambda i: (0, 0)),
        ],
        out_specs=pl.BlockSpec((blk, 2 * hidden), lambda i: (i, 0)),
        out_shape=jax.ShapeDtypeStruct((n, 2 * hidden), jnp.float32),
        scratch_shapes=[
            pltpu.VMEM((1, hidden), jnp.float32),
            pltpu.SMEM((1,), jnp.int32),
        ],
        compiler_params=pltpu.CompilerParams(
            dimension_semantics=("arbitrary",)),
    )(point_features, inv2, wt, g2, b2)

    out = pl.pallas_call(
        _bwd_kernel,
        grid=(nblk,),
        in_specs=[
            pl.BlockSpec((blk, hidden), lambda i, nb=nblk: (nb - 1 - i, 1)),
            pl.BlockSpec((blk, 1), lambda i, nb=nblk: (nb - 1 - i, 0)),
        ],
        out_specs=pl.BlockSpec((blk, hidden), lambda i, nb=nblk: (nb - 1 - i, 1)),
        out_shape=jax.ShapeDtypeStruct((n, 2 * hidden), jnp.float32),
        input_output_aliases={0: 0},
        scratch_shapes=[
            pltpu.VMEM((1, hidden), jnp.float32),
            pltpu.SMEM((1,), jnp.int32),
        ],
        compiler_params=pltpu.CompilerParams(
            dimension_semantics=("arbitrary",)),
    )(z, inv2)
    return out
